# Initial kernel scaffold; baseline (speedup 1.0000x reference)
import jax
import jax.numpy as jnp
from jax.experimental import pallas as pl


def kernel(tokens, emotions, table, W, b):
    raise NotImplementedError("write your pallas kernel here")


# speedup vs baseline: 1.4183x; 1.4183x over previous
import functools

import jax
import jax.numpy as jnp
from jax import lax
from jax.experimental import pallas as pl
from jax.experimental.pallas import tpu as pltpu
from jax.experimental.pallas import tpu_sc as plsc

_B = 4096
_L = 50
_V = 100000
_D = 128

_NW = 32
_ROWS_W = (_B * _L) // _NW
_CHUNK = 100
_NCHUNK = _ROWS_W // _CHUNK
_BATCH_W = _B // _NW
_BPC = _CHUNK // _L


def _tc_project_table(x, w):
    m = x.shape[0]
    blk = 2000

    def body(x_ref, w_ref, o_ref):
        o_ref[...] = lax.dot_general(
            x_ref[...], w_ref[...], (((1,), (1,)), ((), ())),
            preferred_element_type=jnp.float32)

    return pl.pallas_call(
        body,
        grid=(m // blk,),
        in_specs=[
            pl.BlockSpec((blk, _D), lambda i: (i, 0)),
            pl.BlockSpec((_D, _D), lambda i: (0, 0)),
        ],
        out_specs=pl.BlockSpec((blk, _D), lambda i: (i, 0)),
        out_shape=jax.ShapeDtypeStruct((m, _D), jnp.float32),
    )(x, w)


def _tc_project_emotions(x, w, bias):
    m = x.shape[0]
    blk = 2048

    def body(x_ref, w_ref, b_ref, o_ref):
        o_ref[...] = lax.dot_general(
            x_ref[...], w_ref[...], (((1,), (1,)), ((), ())),
            preferred_element_type=jnp.float32) + b_ref[...]

    return pl.pallas_call(
        body,
        grid=(m // blk,),
        in_specs=[
            pl.BlockSpec((blk, _D), lambda i: (i, 0)),
            pl.BlockSpec((_D, _D), lambda i: (0, 0)),
            pl.BlockSpec((1, _D), lambda i: (0, 0)),
        ],
        out_specs=pl.BlockSpec((blk, _D), lambda i: (i, 0)),
        out_shape=jax.ShapeDtypeStruct((m, _D), jnp.float32),
    )(x, w, bias.reshape(1, _D))


@functools.partial(
    pl.kernel,
    out_type=jax.ShapeDtypeStruct((_NW, _NCHUNK, _CHUNK, _D), jnp.float32),
    mesh=plsc.VectorSubcoreMesh(core_axis_name="c", subcore_axis_name="s"),
    scratch_types=[
        pltpu.VMEM((_NCHUNK, _CHUNK), jnp.int32),
        pltpu.VMEM((_BATCH_W, _D), jnp.float32),
        pltpu.VMEM((_CHUNK, _D), jnp.float32),
        pltpu.SemaphoreType.DMA,
    ],
)
def _sc_gather_add(tok_hbm, emo_hbm, proj_hbm, out_hbm,
                   idx_v, emo_v, rows_v, sem):
    w = lax.axis_index("s") * 2 + lax.axis_index("c")
    pltpu.sync_copy(tok_hbm.at[w], idx_v)
    pltpu.sync_copy(emo_hbm.at[w], emo_v)

    def chunk_body(j, _):
        pltpu.async_copy(proj_hbm.at[idx_v.at[j]], rows_v, sem).wait()
        for bi in range(_BPC):
            lb = j * _BPC + bi
            for k in range(_D // 16):
                e = emo_v[lb, pl.ds(k * 16, 16)]

                def row_body(r, _, bi=bi, k=k, e=e):
                    plsc.addupdate(
                        rows_v.at[bi * _L + r, pl.ds(k * 16, 16)], e)
                    return _

                lax.fori_loop(0, _L, row_body, None)
        pltpu.sync_copy(rows_v, out_hbm.at[w, j])
        return _

    lax.fori_loop(0, _NCHUNK, chunk_body, None)


def kernel(tokens, emotions, table, W, b):
    tokens = tokens.astype(jnp.int32)
    we = W[:, :_D]
    wm = W[:, _D:]

    proj = _tc_project_table(table, we)
    emo_proj = _tc_project_emotions(emotions, wm, b)

    tok3 = tokens.reshape(_NW, _NCHUNK, _CHUNK)
    emo3 = emo_proj.reshape(_NW, _BATCH_W, _D)
    out = _sc_gather_add(tok3, emo3, proj)
    return out.reshape(_B, _L, _D)


# speedup vs baseline: 2.4984x; 1.7616x over previous
import functools

import jax
import jax.numpy as jnp
from jax import lax
from jax.experimental import pallas as pl
from jax.experimental.pallas import tpu as pltpu
from jax.experimental.pallas import tpu_sc as plsc

_B = 4096
_L = 50
_V = 100000
_D = 128

_NW = 32
_ROWS_W = (_B * _L) // _NW
_CHUNK = 100
_NCHUNK = _ROWS_W // _CHUNK
_BATCH_W = _B // _NW
_BPC = _CHUNK // _L


def _tc_project_table(x, w):
    m = x.shape[0]
    blk = 2000

    def body(x_ref, w_ref, o_ref):
        o_ref[...] = lax.dot_general(
            x_ref[...], w_ref[...], (((1,), (1,)), ((), ())),
            preferred_element_type=jnp.float32)

    return pl.pallas_call(
        body,
        grid=(m // blk,),
        in_specs=[
            pl.BlockSpec((blk, _D), lambda i: (i, 0)),
            pl.BlockSpec((_D, _D), lambda i: (0, 0)),
        ],
        out_specs=pl.BlockSpec((blk, _D), lambda i: (i, 0)),
        out_shape=jax.ShapeDtypeStruct((m, _D), jnp.float32),
    )(x, w)


def _tc_project_emotions(x, w, bias):
    m = x.shape[0]
    blk = 2048

    def body(x_ref, w_ref, b_ref, o_ref):
        o_ref[...] = lax.dot_general(
            x_ref[...], w_ref[...], (((1,), (1,)), ((), ())),
            preferred_element_type=jnp.float32) + b_ref[...]

    return pl.pallas_call(
        body,
        grid=(m // blk,),
        in_specs=[
            pl.BlockSpec((blk, _D), lambda i: (i, 0)),
            pl.BlockSpec((_D, _D), lambda i: (0, 0)),
            pl.BlockSpec((1, _D), lambda i: (0, 0)),
        ],
        out_specs=pl.BlockSpec((blk, _D), lambda i: (i, 0)),
        out_shape=jax.ShapeDtypeStruct((m, _D), jnp.float32),
    )(x, w, bias.reshape(1, _D))


_NBUF = 4


@functools.partial(
    pl.kernel,
    out_type=jax.ShapeDtypeStruct((_NW, _NCHUNK, _CHUNK, _D), jnp.float32),
    mesh=plsc.VectorSubcoreMesh(core_axis_name="c", subcore_axis_name="s"),
    scratch_types=[
        pltpu.VMEM((_NCHUNK, _CHUNK), jnp.int32),
        pltpu.VMEM((_BATCH_W, _D), jnp.float32),
        pltpu.VMEM((_NBUF, _CHUNK, _D), jnp.float32),
        pltpu.SemaphoreType.DMA((_NBUF,)),
        pltpu.SemaphoreType.DMA((_NBUF,)),
    ],
)
def _sc_gather_add(tok_hbm, emo_hbm, proj_hbm, out_hbm,
                   idx_v, emo_v, rows_v, gsem, ssem):
    w = lax.axis_index("s") * 2 + lax.axis_index("c")
    pltpu.sync_copy(tok_hbm.at[w], idx_v)
    pltpu.sync_copy(emo_hbm.at[w], emo_v)

    def start_gather(j, s):
        pltpu.async_copy(proj_hbm.at[idx_v.at[j]], rows_v.at[s], gsem.at[s])

    for s in range(_NBUF - 1):
        start_gather(s, s)

    def quad_body(jj, _):
        for s in range(_NBUF):
            j = jj * _NBUF + s
            sn = (s + _NBUF - 1) % _NBUF

            @pl.when(j >= 1)
            def _wait_prev_store():
                pltpu.make_async_copy(
                    rows_v.at[sn], out_hbm.at[w, 0], ssem.at[sn]).wait()

            @pl.when(j + _NBUF - 1 < _NCHUNK)
            def _next_gather():
                start_gather(j + _NBUF - 1, sn)

            pltpu.make_async_copy(
                proj_hbm.at[idx_v.at[j]], rows_v.at[s], gsem.at[s]).wait()

            e = [[emo_v[j * _BPC + bi, pl.ds(k * 16, 16)]
                  for k in range(_D // 16)] for bi in range(_BPC)]

            def row_body(r, _, s=s, e=e):
                for bi in range(_BPC):
                    for k in range(_D // 16):
                        plsc.addupdate(
                            rows_v.at[s, bi * _L + r, pl.ds(k * 16, 16)],
                            e[bi][k])
                return _

            lax.fori_loop(0, _L, row_body, None)
            pltpu.async_copy(rows_v.at[s], out_hbm.at[w, j], ssem.at[s])
        return _

    lax.fori_loop(0, _NCHUNK // _NBUF, quad_body, None)
    pltpu.make_async_copy(
        rows_v.at[_NBUF - 1], out_hbm.at[w, 0], ssem.at[_NBUF - 1]).wait()


def kernel(tokens, emotions, table, W, b):
    tokens = tokens.astype(jnp.int32)
    we = W[:, :_D]
    wm = W[:, _D:]

    proj = _tc_project_table(table, we)
    emo_proj = _tc_project_emotions(emotions, wm, b)

    tok3 = tokens.reshape(_NW, _NCHUNK, _CHUNK)
    emo3 = emo_proj.reshape(_NW, _BATCH_W, _D)
    out = _sc_gather_add(tok3, emo3, proj)
    return out.reshape(_B, _L, _D)
